# baseline (speedup 1.0000x reference)
import functools

import jax
import jax.numpy as jnp
from jax import lax
from jax.experimental import pallas as pl
from jax.experimental.pallas import tpu as pltpu
from jax.experimental.pallas import tpu_sc as plsc

N_NODES = 10000
N_EDGES = 320000

NC = 2
NS = 16
L = 16
NW = NC * NS

K = 128
CHUNKS = 80
EPW = K * CHUNKS
E_PAD = EPW * NW
N_PAD = 10240
ZROWS = N_PAD // NS
OROWS = N_NODES // NS


def _fill_rows(rows, value, F):
    vecs_per_row = F // L
    vec = jnp.full((L,), value, jnp.float32)

    @pl.loop(0, K * vecs_per_row)
    def _(i):
        r = i // vecs_per_row
        c = (i % vecs_per_row) * L
        rows[r, pl.ds(c, L)] = vec


NBUF = 4
NBLK = CHUNKS // (2 * NBUF)


def _edge_pipeline(ysh, acc, sidx, didx, rows, gsA, gsB, ssA, ssB):
    def fire_gather(ci, b, sem):
        pltpu.async_copy(ysh.at[sidx.at[ci]], rows.at[b], sem)

    def wait_gather(b, sem):
        pltpu.make_async_copy(ysh.at[sidx.at[0]], rows.at[b], sem).wait()

    def fire_scatter(ci, b, sem):
        pltpu.async_copy(rows.at[b], acc.at[didx.at[ci]], sem, add=True)

    def wait_scatter(b, sem):
        pltpu.make_async_copy(rows.at[b], acc.at[didx.at[0]], sem).wait()

    for b in range(NBUF):
        fire_gather(b, b, gsA)

    @pl.loop(0, NBLK)
    def _(i):
        c0 = i * 2 * NBUF

        @pl.when(i > 0)
        def _():
            for b in range(NBUF):
                wait_scatter(NBUF + b, ssB)

        for b in range(NBUF):
            fire_gather(c0 + NBUF + b, NBUF + b, gsB)
        for b in range(NBUF):
            wait_gather(b, gsA)
        for b in range(NBUF):
            fire_scatter(c0 + b, b, ssA)
        for b in range(NBUF):
            wait_scatter(b, ssA)

        @pl.when(i < NBLK - 1)
        def _():
            for b in range(NBUF):
                fire_gather(c0 + 2 * NBUF + b, b, gsA)

        for b in range(NBUF):
            wait_gather(NBUF + b, gsB)
        for b in range(NBUF):
            fire_scatter(c0 + NBUF + b, NBUF + b, ssB)

    for b in range(NBUF):
        wait_scatter(NBUF + b, ssB)


def _propagate_body(F, stage_y, *refs):
    if stage_y:
        y, srcr, dstr, out, sidx, didx, rows, acc, ysh, gsA, gsB, ssA, ssB = refs
    else:
        y, srcr, dstr, out, sidx, didx, rows, acc, gsA, gsB, ssA, ssB = refs
        ysh = y
    c = lax.axis_index("c")
    s = lax.axis_index("s")
    wid = c * NS + s

    for b in range(NBUF):
        _fill_rows(rows.at[b], 0.0, F)
    for j in range(ZROWS // K):
        pltpu.async_copy(rows.at[j % NBUF],
                         acc.at[pl.ds(s * ZROWS + j * K, K), :], gsA)
    if stage_y:
        pltpu.async_copy(y.at[pl.ds(s * ZROWS, ZROWS), :],
                         ysh.at[pl.ds(s * ZROWS, ZROWS), :], gsB)
    for j in range(ZROWS // K):
        pltpu.make_async_copy(rows.at[0],
                              acc.at[pl.ds(s * ZROWS, K), :], gsA).wait()
    if stage_y:
        pltpu.make_async_copy(y.at[pl.ds(s * ZROWS, ZROWS), :],
                              ysh.at[pl.ds(s * ZROWS, ZROWS), :], gsB).wait()

    plsc.subcore_barrier()

    pltpu.sync_copy(dstr.at[pl.ds(wid * CHUNKS, CHUNKS), :], didx)
    pltpu.sync_copy(srcr.at[pl.ds(wid * CHUNKS, CHUNKS), :], sidx)
    _edge_pipeline(ysh, acc, sidx, didx, rows, gsA, gsB, ssA, ssB)

    plsc.subcore_barrier()

    pltpu.sync_copy(acc.at[pl.ds(s * ZROWS, ZROWS), :],
                    out.at[c, pl.ds(s * ZROWS, ZROWS), :])


def _make_propagate(F, stage_y):
    mesh = plsc.VectorSubcoreMesh(core_axis_name="c", subcore_axis_name="s")
    scratch = [
        pltpu.VMEM((CHUNKS, K), jnp.int32),
        pltpu.VMEM((CHUNKS, K), jnp.int32),
        pltpu.VMEM((2 * NBUF, K, F), jnp.float32),
        pltpu.VMEM_SHARED((N_PAD, F), jnp.float32),
    ]
    if stage_y:
        scratch.append(pltpu.VMEM_SHARED((N_PAD, F), jnp.float32))
    scratch += [pltpu.SemaphoreType.DMA] * 4
    return pl.kernel(
        functools.partial(_propagate_body, F, stage_y),
        out_type=jax.ShapeDtypeStruct((NC, N_PAD, F), jnp.float32),
        mesh=mesh,
        scratch_types=scratch,
        compiler_params=pltpu.CompilerParams(use_tc_tiling_on_sc=False),
    )


_prop32 = _make_propagate(32, stage_y=True)


def _prop64_split_body(y, srcr, dstr, out, sidx, didx, rows,
                       accL, accH, ysh, gsA, gsB, ssA, ssB):
    c = lax.axis_index("c")
    s = lax.axis_index("s")
    wid = c * NS + s

    for b in range(NBUF):
        _fill_rows(rows.at[b], 0.0, 32)
    for j in range(ZROWS // K):
        pltpu.async_copy(rows.at[j % NBUF],
                         accL.at[pl.ds(s * ZROWS + j * K, K), :], gsA)
        pltpu.async_copy(rows.at[j % NBUF],
                         accH.at[pl.ds(s * ZROWS + j * K, K), :], gsA)
    pltpu.async_copy(y.at[pl.ds(s * ZROWS, ZROWS), pl.ds(0, 32)],
                     ysh.at[pl.ds(s * ZROWS, ZROWS), :], gsB)
    pltpu.sync_copy(dstr.at[pl.ds(wid * CHUNKS, CHUNKS), :], didx)
    pltpu.sync_copy(srcr.at[pl.ds(wid * CHUNKS, CHUNKS), :], sidx)
    for j in range(2 * (ZROWS // K)):
        pltpu.make_async_copy(rows.at[0],
                              accL.at[pl.ds(s * ZROWS, K), :], gsA).wait()
    pltpu.make_async_copy(y.at[pl.ds(s * ZROWS, ZROWS), pl.ds(0, 32)],
                          ysh.at[pl.ds(s * ZROWS, ZROWS), :], gsB).wait()
    plsc.subcore_barrier()

    _edge_pipeline(ysh, accL, sidx, didx, rows, gsA, gsB, ssA, ssB)

    plsc.subcore_barrier()
    pltpu.sync_copy(y.at[pl.ds(s * ZROWS, ZROWS), pl.ds(32, 32)],
                    ysh.at[pl.ds(s * ZROWS, ZROWS), :])
    plsc.subcore_barrier()

    _edge_pipeline(ysh, accH, sidx, didx, rows, gsA, gsB, ssA, ssB)

    plsc.subcore_barrier()
    pltpu.sync_copy(accL.at[pl.ds(s * ZROWS, ZROWS), :],
                    out.at[c, pl.ds(s * ZROWS, ZROWS), pl.ds(0, 32)])
    pltpu.sync_copy(accH.at[pl.ds(s * ZROWS, ZROWS), :],
                    out.at[c, pl.ds(s * ZROWS, ZROWS), pl.ds(32, 32)])


_prop64 = pl.kernel(
    _prop64_split_body,
    out_type=jax.ShapeDtypeStruct((NC, N_PAD, 64), jnp.float32),
    mesh=plsc.VectorSubcoreMesh(core_axis_name="c", subcore_axis_name="s"),
    scratch_types=[
        pltpu.VMEM((CHUNKS, K), jnp.int32),
        pltpu.VMEM((CHUNKS, K), jnp.int32),
        pltpu.VMEM((2 * NBUF, K, 32), jnp.float32),
        pltpu.VMEM_SHARED((N_PAD, 32), jnp.float32),
        pltpu.VMEM_SHARED((N_PAD, 32), jnp.float32),
        pltpu.VMEM_SHARED((N_PAD, 32), jnp.float32),
        pltpu.SemaphoreType.DMA,
        pltpu.SemaphoreType.DMA,
        pltpu.SemaphoreType.DMA,
        pltpu.SemaphoreType.DMA,
    ],
    compiler_params=pltpu.CompilerParams(use_tc_tiling_on_sc=False),
)


VPR = K // L


def _deg_body(dst2, out, didx, accv):
    c = lax.axis_index("c")
    s = lax.axis_index("s")
    wid = c * NS + s

    zero = jnp.zeros((L,), jnp.float32)

    @pl.loop(0, N_PAD // L)
    def _(i):
        accv[pl.ds(i * L, L)] = zero

    pltpu.sync_copy(dst2.at[pl.ds(wid * CHUNKS, CHUNKS), :], didx)
    ones = jnp.ones((L,), jnp.float32)

    @pl.loop(0, EPW // L, unroll=4)
    def _(i):
        dvec = didx[i // VPR, pl.ds((i % VPR) * L, L)]
        plsc.addupdate_scatter(accv, [dvec], ones)

    pltpu.sync_copy(accv, out.at[wid])


_deg_sc = pl.kernel(
    _deg_body,
    out_type=jax.ShapeDtypeStruct((NW, N_PAD), jnp.float32),
    mesh=plsc.VectorSubcoreMesh(core_axis_name="c", subcore_axis_name="s"),
    scratch_types=[
        pltpu.VMEM((CHUNKS, K), jnp.int32),
        pltpu.VMEM((N_PAD,), jnp.float32),
    ],
    compiler_params=pltpu.CompilerParams(use_tc_tiling_on_sc=False,
                                         needs_layout_passes=False),
)


def _prop1_body(ybc, src2, dst2, out, yloc, sidx, didx, accv):
    c = lax.axis_index("c")
    s = lax.axis_index("s")
    wid = c * NS + s

    zero = jnp.zeros((L,), jnp.float32)
    zidx = jnp.zeros((L,), jnp.int32)

    @pl.loop(0, N_PAD // L)
    def _(i):
        accv[pl.ds(i * L, L)] = zero

    pltpu.sync_copy(ybc, yloc)
    pltpu.sync_copy(src2.at[pl.ds(wid * CHUNKS, CHUNKS), :], sidx)
    pltpu.sync_copy(dst2.at[pl.ds(wid * CHUNKS, CHUNKS), :], didx)

    @pl.loop(0, EPW // L, unroll=4)
    def _(i):
        svec = sidx[i // VPR, pl.ds((i % VPR) * L, L)]
        dvec = didx[i // VPR, pl.ds((i % VPR) * L, L)]
        vals = plsc.load_gather(yloc, [svec, zidx])
        plsc.addupdate_scatter(accv, [dvec], vals)

    base = wid * (N_PAD // NW)

    @pl.loop(0, (N_PAD // NW) // L)
    def _(j):
        nv = base + j * L + lax.iota(jnp.int32, L)
        vals = plsc.load_gather(yloc, [nv, zidx])
        cur = accv[pl.ds(base + j * L, L)]
        accv[pl.ds(base + j * L, L)] = cur + vals

    pltpu.sync_copy(accv, out.at[wid])


_prop1 = pl.kernel(
    _prop1_body,
    out_type=jax.ShapeDtypeStruct((NW, N_PAD), jnp.float32),
    mesh=plsc.VectorSubcoreMesh(core_axis_name="c", subcore_axis_name="s"),
    scratch_types=[
        pltpu.VMEM((N_PAD, 8), jnp.float32),
        pltpu.VMEM((CHUNKS, K), jnp.int32),
        pltpu.VMEM((CHUNKS, K), jnp.int32),
        pltpu.VMEM((N_PAD,), jnp.float32),
    ],
    compiler_params=pltpu.CompilerParams(use_tc_tiling_on_sc=False,
                                         needs_layout_passes=False),
)


def _stage_a_body(degp, x, w1, y1_out, dinv_out, dinv1d_out):
    ones = jnp.ones((NW, 1), jnp.float32)
    degc = lax.dot_general(degp[...], ones, (((0,), (0,)), ((), ())),
                           preferred_element_type=jnp.float32)
    deg = degc[0:N_NODES, :] + 1.0
    dinv = lax.rsqrt(deg)
    dinv_out[...] = dinv
    dinv1d_out[...] = lax.rsqrt(jnp.sum(degp[...], axis=0) + 1.0)
    y1_out[0:N_NODES, :] = (x[...] @ w1[...]) * dinv


def _stage_mid_body(pp, y, dinv, b, w, out):
    t = (pp[0, :N_NODES] + pp[1, :N_NODES] + y[0:N_NODES, :]) * dinv[...] + b[...]
    h = jnp.maximum(t, 0.0)
    out[0:N_NODES, :] = (h @ w[...]) * dinv[...]


def _stage_c_body(pp, y, dinv, b, w, out):
    t = (pp[0, :N_NODES] + pp[1, :N_NODES] + y[0:N_NODES, :]) * dinv[...] + b[...]
    h = jnp.maximum(t, 0.0)
    out[0:N_NODES, :] = jnp.broadcast_to((h @ w[...]) * dinv[...], (N_NODES, 8))


def _stage_d_body(pp, dinv1d, b, out):
    out[...] = jnp.sum(pp[...], axis=0) * dinv1d[...] + b[0, 0]


def _tc(body, out_shape, *args):
    return pl.pallas_call(body, out_shape=out_shape)(*args)


def kernel(x, edge_index, W1, b1, W2, b2, W3, b3):
    ei = edge_index.astype(jnp.int32)
    npad = E_PAD - N_EDGES
    src = jnp.concatenate(
        [ei[0], jnp.zeros((npad,), jnp.int32)]).reshape(NW * CHUNKS, K)
    dst = jnp.concatenate(
        [ei[1], jnp.full((npad,), N_NODES, jnp.int32)]).reshape(NW * CHUNKS, K)

    degp = _deg_sc(dst)
    y1, dinv, dinv1d = _tc(
        _stage_a_body,
        (jax.ShapeDtypeStruct((N_PAD, 64), jnp.float32),
         jax.ShapeDtypeStruct((N_NODES, 1), jnp.float32),
         jax.ShapeDtypeStruct((N_PAD,), jnp.float32)),
        degp, x, W1)
    p1 = _prop64(y1, src, dst)
    y2 = _tc(
        _stage_mid_body,
        jax.ShapeDtypeStruct((N_PAD, 32), jnp.float32),
        p1, y1, dinv, b1.reshape(1, 64), W2)
    p2 = _prop32(y2, src, dst)
    y3 = _tc(
        _stage_c_body,
        jax.ShapeDtypeStruct((N_PAD, 8), jnp.float32),
        p2, y2, dinv, b2.reshape(1, 32), W3)
    p3 = _prop1(y3, src, dst)
    out = _tc(
        _stage_d_body,
        jax.ShapeDtypeStruct((N_PAD,), jnp.float32),
        p3, dinv1d, b3.reshape(1, 1))
    return out[:N_NODES]


# speedup vs baseline: 1.0343x; 1.0343x over previous
import functools

import jax
import jax.numpy as jnp
from jax import lax
from jax.experimental import pallas as pl
from jax.experimental.pallas import tpu as pltpu
from jax.experimental.pallas import tpu_sc as plsc

N_NODES = 10000
N_EDGES = 320000

NC = 2
NS = 16
L = 16
NW = NC * NS

K = 128
CHUNKS = 80
EPW = K * CHUNKS
E_PAD = EPW * NW
N_PAD = 10240
ZROWS = N_PAD // NS
OROWS = N_NODES // NS


def _fill_rows(rows, value, F):
    vecs_per_row = F // L
    vec = jnp.full((L,), value, jnp.float32)

    @pl.loop(0, K * vecs_per_row)
    def _(i):
        r = i // vecs_per_row
        c = (i % vecs_per_row) * L
        rows[r, pl.ds(c, L)] = vec


NBUF = 4
NBLK = CHUNKS // (2 * NBUF)


def _edge_pipeline(ysh, acc, sidx, didx, rows, gsA, gsB, ssA, ssB):
    def fire_gather(ci, b, sem):
        pltpu.async_copy(ysh.at[sidx.at[ci]], rows.at[b], sem)

    def wait_gather(b, sem):
        pltpu.make_async_copy(ysh.at[sidx.at[0]], rows.at[b], sem).wait()

    def fire_scatter(ci, b, sem):
        pltpu.async_copy(rows.at[b], acc.at[didx.at[ci]], sem, add=True)

    def wait_scatter(b, sem):
        pltpu.make_async_copy(rows.at[b], acc.at[didx.at[0]], sem).wait()

    for b in range(NBUF):
        fire_gather(b, b, gsA)

    @pl.loop(0, NBLK)
    def _(i):
        c0 = i * 2 * NBUF

        @pl.when(i > 0)
        def _():
            for b in range(NBUF):
                wait_scatter(NBUF + b, ssB)

        for b in range(NBUF):
            fire_gather(c0 + NBUF + b, NBUF + b, gsB)
        for b in range(NBUF):
            wait_gather(b, gsA)
        for b in range(NBUF):
            fire_scatter(c0 + b, b, ssA)
        for b in range(NBUF):
            wait_scatter(b, ssA)

        @pl.when(i < NBLK - 1)
        def _():
            for b in range(NBUF):
                fire_gather(c0 + 2 * NBUF + b, b, gsA)

        for b in range(NBUF):
            wait_gather(NBUF + b, gsB)
        for b in range(NBUF):
            fire_scatter(c0 + NBUF + b, NBUF + b, ssB)

    for b in range(NBUF):
        wait_scatter(NBUF + b, ssB)


def _propagate_body(F, stage_y, *refs):
    if stage_y:
        y, edges, out, sidx, didx, rows, acc, ysh, gsA, gsB, ssA, ssB = refs
    else:
        y, edges, out, sidx, didx, rows, acc, gsA, gsB, ssA, ssB = refs
        ysh = y
    srcr = edges.at[0]
    dstr = edges.at[1]
    c = lax.axis_index("c")
    s = lax.axis_index("s")
    wid = c * NS + s

    for b in range(NBUF):
        _fill_rows(rows.at[b], 0.0, F)
    for j in range(ZROWS // K):
        pltpu.async_copy(rows.at[j % NBUF],
                         acc.at[pl.ds(s * ZROWS + j * K, K), :], gsA)
    if stage_y:
        pltpu.async_copy(y.at[pl.ds(s * ZROWS, ZROWS), :],
                         ysh.at[pl.ds(s * ZROWS, ZROWS), :], gsB)
    for j in range(ZROWS // K):
        pltpu.make_async_copy(rows.at[0],
                              acc.at[pl.ds(s * ZROWS, K), :], gsA).wait()
    if stage_y:
        pltpu.make_async_copy(y.at[pl.ds(s * ZROWS, ZROWS), :],
                              ysh.at[pl.ds(s * ZROWS, ZROWS), :], gsB).wait()

    plsc.subcore_barrier()

    pltpu.sync_copy(dstr.at[pl.ds(wid * CHUNKS, CHUNKS), :], didx)
    pltpu.sync_copy(srcr.at[pl.ds(wid * CHUNKS, CHUNKS), :], sidx)
    _edge_pipeline(ysh, acc, sidx, didx, rows, gsA, gsB, ssA, ssB)

    plsc.subcore_barrier()

    pltpu.sync_copy(acc.at[pl.ds(s * ZROWS, ZROWS), :],
                    out.at[c, pl.ds(s * ZROWS, ZROWS), :])


def _make_propagate(F, stage_y):
    mesh = plsc.VectorSubcoreMesh(core_axis_name="c", subcore_axis_name="s")
    scratch = [
        pltpu.VMEM((CHUNKS, K), jnp.int32),
        pltpu.VMEM((CHUNKS, K), jnp.int32),
        pltpu.VMEM((2 * NBUF, K, F), jnp.float32),
        pltpu.VMEM_SHARED((N_PAD, F), jnp.float32),
    ]
    if stage_y:
        scratch.append(pltpu.VMEM_SHARED((N_PAD, F), jnp.float32))
    scratch += [pltpu.SemaphoreType.DMA] * 4
    return pl.kernel(
        functools.partial(_propagate_body, F, stage_y),
        out_type=jax.ShapeDtypeStruct((NC, N_PAD, F), jnp.float32),
        mesh=mesh,
        scratch_types=scratch,
        compiler_params=pltpu.CompilerParams(use_tc_tiling_on_sc=False),
    )


_prop32 = _make_propagate(32, stage_y=True)


def _prop64_split_body(y, edges, out, sidx, didx, rows,
                       accL, accH, ysh, gsA, gsB, ssA, ssB):
    srcr = edges.at[0]
    dstr = edges.at[1]
    c = lax.axis_index("c")
    s = lax.axis_index("s")
    wid = c * NS + s

    for b in range(NBUF):
        _fill_rows(rows.at[b], 0.0, 32)
    for j in range(ZROWS // K):
        pltpu.async_copy(rows.at[j % NBUF],
                         accL.at[pl.ds(s * ZROWS + j * K, K), :], gsA)
        pltpu.async_copy(rows.at[j % NBUF],
                         accH.at[pl.ds(s * ZROWS + j * K, K), :], gsA)
    pltpu.async_copy(y.at[pl.ds(s * ZROWS, ZROWS), pl.ds(0, 32)],
                     ysh.at[pl.ds(s * ZROWS, ZROWS), :], gsB)
    pltpu.sync_copy(dstr.at[pl.ds(wid * CHUNKS, CHUNKS), :], didx)
    pltpu.sync_copy(srcr.at[pl.ds(wid * CHUNKS, CHUNKS), :], sidx)
    for j in range(2 * (ZROWS // K)):
        pltpu.make_async_copy(rows.at[0],
                              accL.at[pl.ds(s * ZROWS, K), :], gsA).wait()
    pltpu.make_async_copy(y.at[pl.ds(s * ZROWS, ZROWS), pl.ds(0, 32)],
                          ysh.at[pl.ds(s * ZROWS, ZROWS), :], gsB).wait()
    plsc.subcore_barrier()

    _edge_pipeline(ysh, accL, sidx, didx, rows, gsA, gsB, ssA, ssB)

    plsc.subcore_barrier()
    pltpu.sync_copy(y.at[pl.ds(s * ZROWS, ZROWS), pl.ds(32, 32)],
                    ysh.at[pl.ds(s * ZROWS, ZROWS), :])
    plsc.subcore_barrier()

    _edge_pipeline(ysh, accH, sidx, didx, rows, gsA, gsB, ssA, ssB)

    plsc.subcore_barrier()
    pltpu.sync_copy(accL.at[pl.ds(s * ZROWS, ZROWS), :],
                    out.at[c, pl.ds(s * ZROWS, ZROWS), pl.ds(0, 32)])
    pltpu.sync_copy(accH.at[pl.ds(s * ZROWS, ZROWS), :],
                    out.at[c, pl.ds(s * ZROWS, ZROWS), pl.ds(32, 32)])


_prop64 = pl.kernel(
    _prop64_split_body,
    out_type=jax.ShapeDtypeStruct((NC, N_PAD, 64), jnp.float32),
    mesh=plsc.VectorSubcoreMesh(core_axis_name="c", subcore_axis_name="s"),
    scratch_types=[
        pltpu.VMEM((CHUNKS, K), jnp.int32),
        pltpu.VMEM((CHUNKS, K), jnp.int32),
        pltpu.VMEM((2 * NBUF, K, 32), jnp.float32),
        pltpu.VMEM_SHARED((N_PAD, 32), jnp.float32),
        pltpu.VMEM_SHARED((N_PAD, 32), jnp.float32),
        pltpu.VMEM_SHARED((N_PAD, 32), jnp.float32),
        pltpu.SemaphoreType.DMA,
        pltpu.SemaphoreType.DMA,
        pltpu.SemaphoreType.DMA,
        pltpu.SemaphoreType.DMA,
    ],
    compiler_params=pltpu.CompilerParams(use_tc_tiling_on_sc=False),
)


VPR = K // L


def _deg_body(edges, out, didx, accv):
    c = lax.axis_index("c")
    s = lax.axis_index("s")
    wid = c * NS + s

    zero = jnp.zeros((L,), jnp.float32)

    @pl.loop(0, N_PAD // L)
    def _(i):
        accv[pl.ds(i * L, L)] = zero

    pltpu.sync_copy(edges.at[1, pl.ds(wid * CHUNKS, CHUNKS), :], didx)
    ones = jnp.ones((L,), jnp.float32)

    @pl.loop(0, EPW // L, unroll=4)
    def _(i):
        dvec = didx[i // VPR, pl.ds((i % VPR) * L, L)]
        plsc.addupdate_scatter(accv, [dvec], ones)

    pltpu.sync_copy(accv, out.at[wid])


_deg_sc = pl.kernel(
    _deg_body,
    out_type=jax.ShapeDtypeStruct((NW, N_PAD), jnp.float32),
    mesh=plsc.VectorSubcoreMesh(core_axis_name="c", subcore_axis_name="s"),
    scratch_types=[
        pltpu.VMEM((CHUNKS, K), jnp.int32),
        pltpu.VMEM((N_PAD,), jnp.float32),
    ],
    compiler_params=pltpu.CompilerParams(use_tc_tiling_on_sc=False,
                                         needs_layout_passes=False),
)


def _prop1_body(ybc, edges, out, yloc, sidx, didx, accv):
    c = lax.axis_index("c")
    s = lax.axis_index("s")
    wid = c * NS + s

    zero = jnp.zeros((L,), jnp.float32)
    zidx = jnp.zeros((L,), jnp.int32)

    @pl.loop(0, N_PAD // L)
    def _(i):
        accv[pl.ds(i * L, L)] = zero

    pltpu.sync_copy(ybc, yloc)
    pltpu.sync_copy(edges.at[0, pl.ds(wid * CHUNKS, CHUNKS), :], sidx)
    pltpu.sync_copy(edges.at[1, pl.ds(wid * CHUNKS, CHUNKS), :], didx)

    @pl.loop(0, EPW // L, unroll=4)
    def _(i):
        svec = sidx[i // VPR, pl.ds((i % VPR) * L, L)]
        dvec = didx[i // VPR, pl.ds((i % VPR) * L, L)]
        vals = plsc.load_gather(yloc, [svec, zidx])
        plsc.addupdate_scatter(accv, [dvec], vals)

    base = wid * (N_PAD // NW)

    @pl.loop(0, (N_PAD // NW) // L)
    def _(j):
        nv = base + j * L + lax.iota(jnp.int32, L)
        vals = plsc.load_gather(yloc, [nv, zidx])
        cur = accv[pl.ds(base + j * L, L)]
        accv[pl.ds(base + j * L, L)] = cur + vals

    pltpu.sync_copy(accv, out.at[wid])


_prop1 = pl.kernel(
    _prop1_body,
    out_type=jax.ShapeDtypeStruct((NW, N_PAD), jnp.float32),
    mesh=plsc.VectorSubcoreMesh(core_axis_name="c", subcore_axis_name="s"),
    scratch_types=[
        pltpu.VMEM((N_PAD, 1), jnp.float32),
        pltpu.VMEM((CHUNKS, K), jnp.int32),
        pltpu.VMEM((CHUNKS, K), jnp.int32),
        pltpu.VMEM((N_PAD,), jnp.float32),
    ],
    compiler_params=pltpu.CompilerParams(use_tc_tiling_on_sc=False,
                                         needs_layout_passes=False),
)


def _stage_a_body(degp, x, w1, y1_out, dinv_out, dinv1d_out):
    ones = jnp.ones((NW, 1), jnp.float32)
    degc = lax.dot_general(degp[...], ones, (((0,), (0,)), ((), ())),
                           preferred_element_type=jnp.float32)
    deg = degc[0:N_NODES, :] + 1.0
    dinv = lax.rsqrt(deg)
    dinv_out[...] = dinv
    dinv1d_out[...] = lax.rsqrt(jnp.sum(degp[...], axis=0) + 1.0)
    y1_out[0:N_NODES, :] = (x[...] @ w1[...]) * dinv


def _stage_mid_body(pp, y, dinv, b, w, out):
    t = (pp[0, :N_NODES] + pp[1, :N_NODES] + y[0:N_NODES, :]) * dinv[...] + b[...]
    h = jnp.maximum(t, 0.0)
    out[0:N_NODES, :] = (h @ w[...]) * dinv[...]


def _stage_c_body(pp, y, dinv, b, w, out):
    t = (pp[0, :N_NODES] + pp[1, :N_NODES] + y[0:N_NODES, :]) * dinv[...] + b[...]
    h = jnp.maximum(t, 0.0)
    out[0:N_NODES, :] = (h @ w[...]) * dinv[...]


def _stage_d_body(pp, dinv1d, b, out):
    out[...] = jnp.sum(pp[...], axis=0) * dinv1d[...] + b[0, 0]


def _tc(body, out_shape, *args):
    return pl.pallas_call(body, out_shape=out_shape)(*args)


def kernel(x, edge_index, W1, b1, W2, b2, W3, b3):
    ei = edge_index.astype(jnp.int32)
    npad = E_PAD - N_EDGES
    pad_block = jnp.concatenate(
        [jnp.zeros((1, npad), jnp.int32),
         jnp.full((1, npad), N_NODES, jnp.int32)], axis=0)
    edges = jnp.concatenate([ei, pad_block], axis=1).reshape(2, NW * CHUNKS, K)

    degp = _deg_sc(edges)
    y1, dinv, dinv1d = _tc(
        _stage_a_body,
        (jax.ShapeDtypeStruct((N_PAD, 64), jnp.float32),
         jax.ShapeDtypeStruct((N_NODES, 1), jnp.float32),
         jax.ShapeDtypeStruct((N_PAD,), jnp.float32)),
        degp, x, W1)
    p1 = _prop64(y1, edges)
    y2 = _tc(
        _stage_mid_body,
        jax.ShapeDtypeStruct((N_PAD, 32), jnp.float32),
        p1, y1, dinv, b1.reshape(1, 64), W2)
    p2 = _prop32(y2, edges)
    y3 = _tc(
        _stage_c_body,
        jax.ShapeDtypeStruct((N_PAD, 1), jnp.float32),
        p2, y2, dinv, b2.reshape(1, 32), W3)
    p3 = _prop1(y3, edges)
    out = _tc(
        _stage_d_body,
        jax.ShapeDtypeStruct((N_PAD,), jnp.float32),
        p3, dinv1d, b3.reshape(1, 1))
    return out[:N_NODES]


# speedup vs baseline: 1.0369x; 1.0025x over previous
import functools

import jax
import jax.numpy as jnp
from jax import lax
from jax.experimental import pallas as pl
from jax.experimental.pallas import tpu as pltpu
from jax.experimental.pallas import tpu_sc as plsc

N_NODES = 10000
N_EDGES = 320000

NC = 2
NS = 16
L = 16
NW = NC * NS

K = 128
CHUNKS = 80
EPW = K * CHUNKS
E_PAD = EPW * NW
N_PAD = 10240
ZROWS = N_PAD // NS
OROWS = N_NODES // NS


def _fill_rows(rows, value, F):
    vecs_per_row = F // L
    vec = jnp.full((L,), value, jnp.float32)

    @pl.loop(0, K * vecs_per_row)
    def _(i):
        r = i // vecs_per_row
        c = (i % vecs_per_row) * L
        rows[r, pl.ds(c, L)] = vec


NBUF = 4
NBLK = CHUNKS // (2 * NBUF)


def _edge_pipeline(ysh, acc, sidx, didx, rows, gsA, gsB, ssA, ssB):
    def fire_gather(ci, b, sem):
        pltpu.async_copy(ysh.at[sidx.at[ci]], rows.at[b], sem)

    def wait_gather(b, sem):
        pltpu.make_async_copy(ysh.at[sidx.at[0]], rows.at[b], sem).wait()

    def fire_scatter(ci, b, sem):
        pltpu.async_copy(rows.at[b], acc.at[didx.at[ci]], sem, add=True)

    def wait_scatter(b, sem):
        pltpu.make_async_copy(rows.at[b], acc.at[didx.at[0]], sem).wait()

    for b in range(NBUF):
        fire_gather(b, b, gsA)

    @pl.loop(0, NBLK)
    def _(i):
        c0 = i * 2 * NBUF

        @pl.when(i > 0)
        def _():
            for b in range(NBUF):
                wait_scatter(NBUF + b, ssB)

        for b in range(NBUF):
            fire_gather(c0 + NBUF + b, NBUF + b, gsB)
        for b in range(NBUF):
            wait_gather(b, gsA)
        for b in range(NBUF):
            fire_scatter(c0 + b, b, ssA)
        for b in range(NBUF):
            wait_scatter(b, ssA)

        @pl.when(i < NBLK - 1)
        def _():
            for b in range(NBUF):
                fire_gather(c0 + 2 * NBUF + b, b, gsA)

        for b in range(NBUF):
            wait_gather(NBUF + b, gsB)
        for b in range(NBUF):
            fire_scatter(c0 + NBUF + b, NBUF + b, ssB)

    for b in range(NBUF):
        wait_scatter(NBUF + b, ssB)


def _propagate_body(F, stage_y, *refs):
    if stage_y:
        y, edges, out, sidx, didx, rows, acc, ysh, gsA, gsB, ssA, ssB = refs
    else:
        y, edges, out, sidx, didx, rows, acc, gsA, gsB, ssA, ssB = refs
        ysh = y
    srcr = edges.at[0]
    dstr = edges.at[1]
    c = lax.axis_index("c")
    s = lax.axis_index("s")
    wid = c * NS + s

    for b in range(NBUF):
        _fill_rows(rows.at[b], 0.0, F)
    for j in range(ZROWS // K):
        pltpu.async_copy(rows.at[j % NBUF],
                         acc.at[pl.ds(s * ZROWS + j * K, K), :], gsA)
    if stage_y:
        pltpu.async_copy(y.at[pl.ds(s * ZROWS, ZROWS), :],
                         ysh.at[pl.ds(s * ZROWS, ZROWS), :], gsB)
    for j in range(ZROWS // K):
        pltpu.make_async_copy(rows.at[0],
                              acc.at[pl.ds(s * ZROWS, K), :], gsA).wait()
    if stage_y:
        pltpu.make_async_copy(y.at[pl.ds(s * ZROWS, ZROWS), :],
                              ysh.at[pl.ds(s * ZROWS, ZROWS), :], gsB).wait()

    plsc.subcore_barrier()

    pltpu.sync_copy(dstr.at[pl.ds(wid * CHUNKS, CHUNKS), :], didx)
    pltpu.sync_copy(srcr.at[pl.ds(wid * CHUNKS, CHUNKS), :], sidx)
    _edge_pipeline(ysh, acc, sidx, didx, rows, gsA, gsB, ssA, ssB)

    plsc.subcore_barrier()

    pltpu.sync_copy(acc.at[pl.ds(s * ZROWS, ZROWS), :],
                    out.at[c, pl.ds(s * ZROWS, ZROWS), :])


def _make_propagate(F, stage_y):
    mesh = plsc.VectorSubcoreMesh(core_axis_name="c", subcore_axis_name="s")
    scratch = [
        pltpu.VMEM((CHUNKS, K), jnp.int32),
        pltpu.VMEM((CHUNKS, K), jnp.int32),
        pltpu.VMEM((2 * NBUF, K, F), jnp.float32),
        pltpu.VMEM_SHARED((N_PAD, F), jnp.float32),
    ]
    if stage_y:
        scratch.append(pltpu.VMEM_SHARED((N_PAD, F), jnp.float32))
    scratch += [pltpu.SemaphoreType.DMA] * 4
    return pl.kernel(
        functools.partial(_propagate_body, F, stage_y),
        out_type=jax.ShapeDtypeStruct((NC, N_PAD, F), jnp.float32),
        mesh=mesh,
        scratch_types=scratch,
        compiler_params=pltpu.CompilerParams(use_tc_tiling_on_sc=False),
    )


_prop32 = _make_propagate(32, stage_y=True)


def _prop64_split_body(y, edges, out, sidx, didx, rows,
                       accL, accH, ysh, gsA, gsB, ssA, ssB):
    srcr = edges.at[0]
    dstr = edges.at[1]
    c = lax.axis_index("c")
    s = lax.axis_index("s")
    wid = c * NS + s

    for b in range(NBUF):
        _fill_rows(rows.at[b], 0.0, 32)
    for j in range(ZROWS // K):
        pltpu.async_copy(rows.at[j % NBUF],
                         accL.at[pl.ds(s * ZROWS + j * K, K), :], gsA)
        pltpu.async_copy(rows.at[j % NBUF],
                         accH.at[pl.ds(s * ZROWS + j * K, K), :], gsA)
    pltpu.async_copy(y.at[pl.ds(s * ZROWS, ZROWS), pl.ds(0, 32)],
                     ysh.at[pl.ds(s * ZROWS, ZROWS), :], gsB)
    pltpu.sync_copy(dstr.at[pl.ds(wid * CHUNKS, CHUNKS), :], didx)
    pltpu.sync_copy(srcr.at[pl.ds(wid * CHUNKS, CHUNKS), :], sidx)
    for j in range(2 * (ZROWS // K)):
        pltpu.make_async_copy(rows.at[0],
                              accL.at[pl.ds(s * ZROWS, K), :], gsA).wait()
    pltpu.make_async_copy(y.at[pl.ds(s * ZROWS, ZROWS), pl.ds(0, 32)],
                          ysh.at[pl.ds(s * ZROWS, ZROWS), :], gsB).wait()
    plsc.subcore_barrier()

    _edge_pipeline(ysh, accL, sidx, didx, rows, gsA, gsB, ssA, ssB)

    plsc.subcore_barrier()
    pltpu.sync_copy(y.at[pl.ds(s * ZROWS, ZROWS), pl.ds(32, 32)],
                    ysh.at[pl.ds(s * ZROWS, ZROWS), :])
    plsc.subcore_barrier()

    _edge_pipeline(ysh, accH, sidx, didx, rows, gsA, gsB, ssA, ssB)

    plsc.subcore_barrier()
    pltpu.sync_copy(accL.at[pl.ds(s * ZROWS, ZROWS), :],
                    out.at[c, pl.ds(s * ZROWS, ZROWS), pl.ds(0, 32)])
    pltpu.sync_copy(accH.at[pl.ds(s * ZROWS, ZROWS), :],
                    out.at[c, pl.ds(s * ZROWS, ZROWS), pl.ds(32, 32)])


_prop64 = pl.kernel(
    _prop64_split_body,
    out_type=jax.ShapeDtypeStruct((NC, N_PAD, 64), jnp.float32),
    mesh=plsc.VectorSubcoreMesh(core_axis_name="c", subcore_axis_name="s"),
    scratch_types=[
        pltpu.VMEM((CHUNKS, K), jnp.int32),
        pltpu.VMEM((CHUNKS, K), jnp.int32),
        pltpu.VMEM((2 * NBUF, K, 32), jnp.float32),
        pltpu.VMEM_SHARED((N_PAD, 32), jnp.float32),
        pltpu.VMEM_SHARED((N_PAD, 32), jnp.float32),
        pltpu.VMEM_SHARED((N_PAD, 32), jnp.float32),
        pltpu.SemaphoreType.DMA,
        pltpu.SemaphoreType.DMA,
        pltpu.SemaphoreType.DMA,
        pltpu.SemaphoreType.DMA,
    ],
    compiler_params=pltpu.CompilerParams(use_tc_tiling_on_sc=False),
)


VPR = K // L


def _deg_body(edges, out, didx, accv):
    c = lax.axis_index("c")
    s = lax.axis_index("s")
    wid = c * NS + s

    zero = jnp.zeros((L,), jnp.float32)

    @pl.loop(0, N_PAD // L)
    def _(i):
        accv[pl.ds(i * L, L)] = zero

    pltpu.sync_copy(edges.at[1, pl.ds(wid * CHUNKS, CHUNKS), :], didx)
    ones = jnp.ones((L,), jnp.float32)

    @pl.loop(0, CHUNKS)
    def _(r):
        for v in range(VPR):
            dvec = didx[r, pl.ds(v * L, L)]
            plsc.addupdate_scatter(accv, [dvec], ones)

    pltpu.sync_copy(accv, out.at[wid])


_deg_sc = pl.kernel(
    _deg_body,
    out_type=jax.ShapeDtypeStruct((NW, N_PAD), jnp.float32),
    mesh=plsc.VectorSubcoreMesh(core_axis_name="c", subcore_axis_name="s"),
    scratch_types=[
        pltpu.VMEM((CHUNKS, K), jnp.int32),
        pltpu.VMEM((N_PAD,), jnp.float32),
    ],
    compiler_params=pltpu.CompilerParams(use_tc_tiling_on_sc=False,
                                         needs_layout_passes=False),
)


def _prop1_body(ybc, edges, out, yloc, sidx, didx, accv):
    c = lax.axis_index("c")
    s = lax.axis_index("s")
    wid = c * NS + s

    zero = jnp.zeros((L,), jnp.float32)
    zidx = jnp.zeros((L,), jnp.int32)

    @pl.loop(0, N_PAD // L)
    def _(i):
        accv[pl.ds(i * L, L)] = zero

    pltpu.sync_copy(ybc, yloc)
    pltpu.sync_copy(edges.at[0, pl.ds(wid * CHUNKS, CHUNKS), :], sidx)
    pltpu.sync_copy(edges.at[1, pl.ds(wid * CHUNKS, CHUNKS), :], didx)

    @pl.loop(0, CHUNKS)
    def _(r):
        for v in range(VPR):
            svec = sidx[r, pl.ds(v * L, L)]
            dvec = didx[r, pl.ds(v * L, L)]
            vals = plsc.load_gather(yloc, [svec, zidx])
            plsc.addupdate_scatter(accv, [dvec], vals)

    base = wid * (N_PAD // NW)

    @pl.loop(0, (N_PAD // NW) // L)
    def _(j):
        nv = base + j * L + lax.iota(jnp.int32, L)
        vals = plsc.load_gather(yloc, [nv, zidx])
        cur = accv[pl.ds(base + j * L, L)]
        accv[pl.ds(base + j * L, L)] = cur + vals

    pltpu.sync_copy(accv, out.at[wid])


_prop1 = pl.kernel(
    _prop1_body,
    out_type=jax.ShapeDtypeStruct((NW, N_PAD), jnp.float32),
    mesh=plsc.VectorSubcoreMesh(core_axis_name="c", subcore_axis_name="s"),
    scratch_types=[
        pltpu.VMEM((N_PAD, 1), jnp.float32),
        pltpu.VMEM((CHUNKS, K), jnp.int32),
        pltpu.VMEM((CHUNKS, K), jnp.int32),
        pltpu.VMEM((N_PAD,), jnp.float32),
    ],
    compiler_params=pltpu.CompilerParams(use_tc_tiling_on_sc=False,
                                         needs_layout_passes=False),
)


def _stage_a_body(degp, x, w1, y1_out, dinv_out, dinv1d_out):
    ones = jnp.ones((NW, 1), jnp.float32)
    degc = lax.dot_general(degp[...], ones, (((0,), (0,)), ((), ())),
                           preferred_element_type=jnp.float32)
    deg = degc[0:N_NODES, :] + 1.0
    dinv = lax.rsqrt(deg)
    dinv_out[...] = dinv
    dinv1d_out[...] = lax.rsqrt(jnp.sum(degp[...], axis=0) + 1.0)
    y1_out[0:N_NODES, :] = (x[...] @ w1[...]) * dinv


def _stage_mid_body(pp, y, dinv, b, w, out):
    t = (pp[0, :N_NODES] + pp[1, :N_NODES] + y[0:N_NODES, :]) * dinv[...] + b[...]
    h = jnp.maximum(t, 0.0)
    out[0:N_NODES, :] = (h @ w[...]) * dinv[...]


def _stage_c_body(pp, y, dinv, b, w, out):
    t = (pp[0, :N_NODES] + pp[1, :N_NODES] + y[0:N_NODES, :]) * dinv[...] + b[...]
    h = jnp.maximum(t, 0.0)
    out[0:N_NODES, :] = (h @ w[...]) * dinv[...]


def _stage_d_body(pp, dinv1d, b, out):
    out[...] = jnp.sum(pp[...], axis=0) * dinv1d[...] + b[0, 0]


def _tc(body, out_shape, *args):
    return pl.pallas_call(body, out_shape=out_shape)(*args)


def kernel(x, edge_index, W1, b1, W2, b2, W3, b3):
    ei = edge_index.astype(jnp.int32)
    npad = E_PAD - N_EDGES
    pad_block = jnp.concatenate(
        [jnp.zeros((1, npad), jnp.int32),
         jnp.full((1, npad), N_NODES, jnp.int32)], axis=0)
    edges = jnp.concatenate([ei, pad_block], axis=1).reshape(2, NW * CHUNKS, K)

    degp = _deg_sc(edges)
    y1, dinv, dinv1d = _tc(
        _stage_a_body,
        (jax.ShapeDtypeStruct((N_PAD, 64), jnp.float32),
         jax.ShapeDtypeStruct((N_NODES, 1), jnp.float32),
         jax.ShapeDtypeStruct((N_PAD,), jnp.float32)),
        degp, x, W1)
    p1 = _prop64(y1, edges)
    y2 = _tc(
        _stage_mid_body,
        jax.ShapeDtypeStruct((N_PAD, 32), jnp.float32),
        p1, y1, dinv, b1.reshape(1, 64), W2)
    p2 = _prop32(y2, edges)
    y3 = _tc(
        _stage_c_body,
        jax.ShapeDtypeStruct((N_PAD, 1), jnp.float32),
        p2, y2, dinv, b2.reshape(1, 32), W3)
    p3 = _prop1(y3, edges)
    out = _tc(
        _stage_d_body,
        jax.ShapeDtypeStruct((N_PAD,), jnp.float32),
        p3, dinv1d, b3.reshape(1, 1))
    return out[:N_NODES]


# speedup vs baseline: 1.0849x; 1.0463x over previous
import functools

import jax
import jax.numpy as jnp
from jax import lax
from jax.experimental import pallas as pl
from jax.experimental.pallas import tpu as pltpu
from jax.experimental.pallas import tpu_sc as plsc

N_NODES = 10000
N_EDGES = 320000

NC = 2
NS = 16
L = 16
NW = NC * NS

K = 128
CHUNKS = 80
EPW = K * CHUNKS
E_PAD = EPW * NW
N_PAD = 10240
ZROWS = N_PAD // NS
OROWS = N_NODES // NS


def _fill_rows(rows, value, F):
    vecs_per_row = F // L
    vec = jnp.full((L,), value, jnp.float32)

    @pl.loop(0, K * vecs_per_row)
    def _(i):
        r = i // vecs_per_row
        c = (i % vecs_per_row) * L
        rows[r, pl.ds(c, L)] = vec


NBUF = 4
NBLK = CHUNKS // (2 * NBUF)


def _edge_pipeline(ysh, acc, sidx, didx, rows, gsA, gsB, ssA, ssB):
    def fire_gather(ci, b, sem):
        pltpu.async_copy(ysh.at[sidx.at[ci]], rows.at[b], sem)

    def wait_gather(b, sem):
        pltpu.make_async_copy(ysh.at[sidx.at[0]], rows.at[b], sem).wait()

    def fire_scatter(ci, b, sem):
        pltpu.async_copy(rows.at[b], acc.at[didx.at[ci]], sem, add=True)

    def wait_scatter(b, sem):
        pltpu.make_async_copy(rows.at[b], acc.at[didx.at[0]], sem).wait()

    for b in range(NBUF):
        fire_gather(b, b, gsA)

    @pl.loop(0, NBLK)
    def _(i):
        c0 = i * 2 * NBUF

        @pl.when(i > 0)
        def _():
            for b in range(NBUF):
                wait_scatter(NBUF + b, ssB)

        for b in range(NBUF):
            fire_gather(c0 + NBUF + b, NBUF + b, gsB)
        for b in range(NBUF):
            wait_gather(b, gsA)
        for b in range(NBUF):
            fire_scatter(c0 + b, b, ssA)
        for b in range(NBUF):
            wait_scatter(b, ssA)

        @pl.when(i < NBLK - 1)
        def _():
            for b in range(NBUF):
                fire_gather(c0 + 2 * NBUF + b, b, gsA)

        for b in range(NBUF):
            wait_gather(NBUF + b, gsB)
        for b in range(NBUF):
            fire_scatter(c0 + NBUF + b, NBUF + b, ssB)

    for b in range(NBUF):
        wait_scatter(NBUF + b, ssB)


def _propagate_body(F, stage_y, *refs):
    if stage_y:
        y, edges, out, sidx, didx, rows, acc, ysh, gsA, gsB, ssA, ssB = refs
    else:
        y, edges, out, sidx, didx, rows, acc, gsA, gsB, ssA, ssB = refs
        ysh = y
    srcr = edges.at[0]
    dstr = edges.at[1]
    c = lax.axis_index("c")
    s = lax.axis_index("s")
    wid = c * NS + s

    for b in range(NBUF):
        _fill_rows(rows.at[b], 0.0, F)
    for j in range(ZROWS // K):
        pltpu.async_copy(rows.at[j % NBUF],
                         acc.at[pl.ds(s * ZROWS + j * K, K), :], gsA)
    if stage_y:
        pltpu.async_copy(y.at[pl.ds(s * ZROWS, ZROWS), :],
                         ysh.at[pl.ds(s * ZROWS, ZROWS), :], gsB)
    for j in range(ZROWS // K):
        pltpu.make_async_copy(rows.at[0],
                              acc.at[pl.ds(s * ZROWS, K), :], gsA).wait()
    if stage_y:
        pltpu.make_async_copy(y.at[pl.ds(s * ZROWS, ZROWS), :],
                              ysh.at[pl.ds(s * ZROWS, ZROWS), :], gsB).wait()

    plsc.subcore_barrier()

    pltpu.sync_copy(dstr.at[pl.ds(wid * CHUNKS, CHUNKS), :], didx)
    pltpu.sync_copy(srcr.at[pl.ds(wid * CHUNKS, CHUNKS), :], sidx)
    _edge_pipeline(ysh, acc, sidx, didx, rows, gsA, gsB, ssA, ssB)

    plsc.subcore_barrier()

    pltpu.sync_copy(acc.at[pl.ds(s * ZROWS, ZROWS), :],
                    out.at[c, pl.ds(s * ZROWS, ZROWS), :])


def _make_propagate(F, stage_y):
    mesh = plsc.VectorSubcoreMesh(core_axis_name="c", subcore_axis_name="s")
    scratch = [
        pltpu.VMEM((CHUNKS, K), jnp.int32),
        pltpu.VMEM((CHUNKS, K), jnp.int32),
        pltpu.VMEM((2 * NBUF, K, F), jnp.float32),
        pltpu.VMEM_SHARED((N_PAD, F), jnp.float32),
    ]
    if stage_y:
        scratch.append(pltpu.VMEM_SHARED((N_PAD, F), jnp.float32))
    scratch += [pltpu.SemaphoreType.DMA] * 4
    return pl.kernel(
        functools.partial(_propagate_body, F, stage_y),
        out_type=jax.ShapeDtypeStruct((NC, N_PAD, F), jnp.float32),
        mesh=mesh,
        scratch_types=scratch,
        compiler_params=pltpu.CompilerParams(use_tc_tiling_on_sc=False),
    )


_prop32 = _make_propagate(32, stage_y=True)


def _prop64_split_body(y, edges, out, sidx, didx, rows,
                       accL, accH, ysh, gsA, gsB, ssA, ssB):
    srcr = edges.at[0]
    dstr = edges.at[1]
    c = lax.axis_index("c")
    s = lax.axis_index("s")
    wid = c * NS + s

    for b in range(NBUF):
        _fill_rows(rows.at[b], 0.0, 32)
    for j in range(ZROWS // K):
        pltpu.async_copy(rows.at[j % NBUF],
                         accL.at[pl.ds(s * ZROWS + j * K, K), :], gsA)
        pltpu.async_copy(rows.at[j % NBUF],
                         accH.at[pl.ds(s * ZROWS + j * K, K), :], gsA)
    pltpu.async_copy(y.at[pl.ds(s * ZROWS, ZROWS), pl.ds(0, 32)],
                     ysh.at[pl.ds(s * ZROWS, ZROWS), :], gsB)
    pltpu.sync_copy(dstr.at[pl.ds(wid * CHUNKS, CHUNKS), :], didx)
    pltpu.sync_copy(srcr.at[pl.ds(wid * CHUNKS, CHUNKS), :], sidx)
    for j in range(2 * (ZROWS // K)):
        pltpu.make_async_copy(rows.at[0],
                              accL.at[pl.ds(s * ZROWS, K), :], gsA).wait()
    pltpu.make_async_copy(y.at[pl.ds(s * ZROWS, ZROWS), pl.ds(0, 32)],
                          ysh.at[pl.ds(s * ZROWS, ZROWS), :], gsB).wait()
    plsc.subcore_barrier()

    _edge_pipeline(ysh, accL, sidx, didx, rows, gsA, gsB, ssA, ssB)

    plsc.subcore_barrier()
    pltpu.sync_copy(y.at[pl.ds(s * ZROWS, ZROWS), pl.ds(32, 32)],
                    ysh.at[pl.ds(s * ZROWS, ZROWS), :])
    plsc.subcore_barrier()

    _edge_pipeline(ysh, accH, sidx, didx, rows, gsA, gsB, ssA, ssB)

    plsc.subcore_barrier()
    pltpu.sync_copy(accL.at[pl.ds(s * ZROWS, ZROWS), :],
                    out.at[c, pl.ds(s * ZROWS, ZROWS), pl.ds(0, 32)])
    pltpu.sync_copy(accH.at[pl.ds(s * ZROWS, ZROWS), :],
                    out.at[c, pl.ds(s * ZROWS, ZROWS), pl.ds(32, 32)])


_prop64 = pl.kernel(
    _prop64_split_body,
    out_type=jax.ShapeDtypeStruct((NC, N_PAD, 64), jnp.float32),
    mesh=plsc.VectorSubcoreMesh(core_axis_name="c", subcore_axis_name="s"),
    scratch_types=[
        pltpu.VMEM((CHUNKS, K), jnp.int32),
        pltpu.VMEM((CHUNKS, K), jnp.int32),
        pltpu.VMEM((2 * NBUF, K, 32), jnp.float32),
        pltpu.VMEM_SHARED((N_PAD, 32), jnp.float32),
        pltpu.VMEM_SHARED((N_PAD, 32), jnp.float32),
        pltpu.VMEM_SHARED((N_PAD, 32), jnp.float32),
        pltpu.SemaphoreType.DMA,
        pltpu.SemaphoreType.DMA,
        pltpu.SemaphoreType.DMA,
        pltpu.SemaphoreType.DMA,
    ],
    compiler_params=pltpu.CompilerParams(use_tc_tiling_on_sc=False),
)


VPR = K // L


def _deg_body(edges, out, didx, accv):
    c = lax.axis_index("c")
    s = lax.axis_index("s")
    wid = c * NS + s

    zero = jnp.zeros((L,), jnp.float32)

    @pl.loop(0, N_PAD // L)
    def _(i):
        accv[pl.ds(i * L, L)] = zero

    pltpu.sync_copy(edges.at[1, pl.ds(wid * CHUNKS, CHUNKS), :], didx)
    ones = jnp.ones((L,), jnp.float32)

    @pl.loop(0, CHUNKS)
    def _(r):
        for v in range(VPR):
            dvec = didx[r, pl.ds(v * L, L)]
            plsc.addupdate_scatter(accv, [dvec], ones)

    pltpu.sync_copy(accv, out.at[wid])


_deg_sc = pl.kernel(
    _deg_body,
    out_type=jax.ShapeDtypeStruct((NW, N_PAD), jnp.float32),
    mesh=plsc.VectorSubcoreMesh(core_axis_name="c", subcore_axis_name="s"),
    scratch_types=[
        pltpu.VMEM((CHUNKS, K), jnp.int32),
        pltpu.VMEM((N_PAD,), jnp.float32),
    ],
    compiler_params=pltpu.CompilerParams(use_tc_tiling_on_sc=False,
                                         needs_layout_passes=False),
)


def _prop1_body(ybc, edges, out, yloc, sidx, didx, accv):
    c = lax.axis_index("c")
    s = lax.axis_index("s")
    wid = c * NS + s

    zero = jnp.zeros((L,), jnp.float32)

    @pl.loop(0, N_PAD // L)
    def _(i):
        accv[pl.ds(i * L, L)] = zero

    pltpu.sync_copy(ybc, yloc)
    pltpu.sync_copy(edges.at[0, pl.ds(wid * CHUNKS, CHUNKS), :], sidx)
    pltpu.sync_copy(edges.at[1, pl.ds(wid * CHUNKS, CHUNKS), :], didx)

    @pl.loop(0, CHUNKS)
    def _(r):
        for v in range(VPR):
            svec = sidx[r, pl.ds(v * L, L)]
            dvec = didx[r, pl.ds(v * L, L)]
            vals = plsc.load_gather(yloc, [svec])
            plsc.addupdate_scatter(accv, [dvec], vals)

    base = wid * (N_PAD // NW)

    @pl.loop(0, (N_PAD // NW) // L)
    def _(j):
        off = base + j * L
        cur = accv[pl.ds(off, L)]
        accv[pl.ds(off, L)] = cur + yloc[pl.ds(off, L)]

    pltpu.sync_copy(accv, out.at[wid])


_prop1 = pl.kernel(
    _prop1_body,
    out_type=jax.ShapeDtypeStruct((NW, N_PAD), jnp.float32),
    mesh=plsc.VectorSubcoreMesh(core_axis_name="c", subcore_axis_name="s"),
    scratch_types=[
        pltpu.VMEM((N_PAD,), jnp.float32),
        pltpu.VMEM((CHUNKS, K), jnp.int32),
        pltpu.VMEM((CHUNKS, K), jnp.int32),
        pltpu.VMEM((N_PAD,), jnp.float32),
    ],
    compiler_params=pltpu.CompilerParams(use_tc_tiling_on_sc=False,
                                         needs_layout_passes=False),
)


def _stage_a_body(degp, x, w1, y1_out, dinv_out, dinv1d_out):
    ones = jnp.ones((NW, 1), jnp.float32)
    degc = lax.dot_general(degp[...], ones, (((0,), (0,)), ((), ())),
                           preferred_element_type=jnp.float32)
    deg = degc[0:N_NODES, :] + 1.0
    dinv = lax.rsqrt(deg)
    dinv_out[...] = dinv
    dinv1d_out[...] = lax.rsqrt(jnp.sum(degp[...], axis=0) + 1.0)
    y1_out[0:N_NODES, :] = (x[...] @ w1[...]) * dinv


def _stage_mid_body(pp, y, dinv, b, w, out):
    t = (pp[0, :N_NODES] + pp[1, :N_NODES] + y[0:N_NODES, :]) * dinv[...] + b[...]
    h = jnp.maximum(t, 0.0)
    out[0:N_NODES, :] = (h @ w[...]) * dinv[...]


def _stage_c_body(pp, y, dinv, b, w, out):
    t = (pp[0, :N_NODES] + pp[1, :N_NODES] + y[0:N_NODES, :]) * dinv[...] + b[...]
    h = jnp.maximum(t, 0.0)
    out[0:N_NODES, :] = (h @ w[...]) * dinv[...]


def _stage_d_body(pp, dinv1d, b, out):
    out[...] = jnp.sum(pp[...], axis=0) * dinv1d[...] + b[0, 0]


def _tc(body, out_shape, *args):
    return pl.pallas_call(body, out_shape=out_shape)(*args)


def kernel(x, edge_index, W1, b1, W2, b2, W3, b3):
    ei = edge_index.astype(jnp.int32)
    npad = E_PAD - N_EDGES
    pad_block = jnp.concatenate(
        [jnp.zeros((1, npad), jnp.int32),
         jnp.full((1, npad), N_NODES, jnp.int32)], axis=0)
    edges = jnp.concatenate([ei, pad_block], axis=1).reshape(2, NW * CHUNKS, K)

    degp = _deg_sc(edges)
    y1, dinv, dinv1d = _tc(
        _stage_a_body,
        (jax.ShapeDtypeStruct((N_PAD, 64), jnp.float32),
         jax.ShapeDtypeStruct((N_NODES, 1), jnp.float32),
         jax.ShapeDtypeStruct((N_PAD,), jnp.float32)),
        degp, x, W1)
    p1 = _prop64(y1, edges)
    y2 = _tc(
        _stage_mid_body,
        jax.ShapeDtypeStruct((N_PAD, 32), jnp.float32),
        p1, y1, dinv, b1.reshape(1, 64), W2)
    p2 = _prop32(y2, edges)
    y3 = _tc(
        _stage_c_body,
        jax.ShapeDtypeStruct((N_PAD, 1), jnp.float32),
        p2, y2, dinv, b2.reshape(1, 32), W3)
    p3 = _prop1(y3.reshape(N_PAD), edges)
    out = _tc(
        _stage_d_body,
        jax.ShapeDtypeStruct((N_PAD,), jnp.float32),
        p3, dinv1d, b3.reshape(1, 1))
    return out[:N_NODES]
